# Initial kernel scaffold; baseline (speedup 1.0000x reference)
import jax
import jax.numpy as jnp
from jax.experimental import pallas as pl


def kernel(coord0, feat0, offset0, coord1, feat1, offset1, cluster, idx_ptr, sorted_cluster_indices, W_u, b_u, bn_u_w, bn_u_b, W_g, bn_g_w, bn_g_b):
    raise NotImplementedError("write your pallas kernel here")


# speedup vs baseline: 1.0408x; 1.0408x over previous
import functools

import jax
import jax.numpy as jnp
from jax import lax
from jax.experimental import pallas as pl
from jax.experimental.pallas import tpu as pltpu
from jax.experimental.pallas import tpu_sc as plsc

N0 = 100000
N1 = 25000
C = 128
EPS = 1e-5

NC = 2
NS = 16
NW = NC * NS

F32 = jnp.float32


def _mmbn_body(T, N, x_ref, w_ref, g_ref, b_ref, o_ref, s_ref, ss_ref,
               sc_ref, sh_ref):
    i = pl.program_id(0)
    z = jnp.dot(x_ref[...], w_ref[...], preferred_element_type=F32)

    @pl.when(i == 0)
    def _():
        s_ref[...] = jnp.zeros_like(s_ref)
        ss_ref[...] = jnp.zeros_like(ss_ref)

    @pl.when(i < T)
    def _():
        s_ref[...] += jnp.sum(z, axis=0, keepdims=True)
        ss_ref[...] += jnp.sum(z * z, axis=0, keepdims=True)

    @pl.when(i == T)
    def _():
        m = s_ref[...] / N
        v = ss_ref[...] / N - m * m
        sc = g_ref[...] * lax.rsqrt(v + EPS)
        sc_ref[...] = sc
        sh_ref[...] = b_ref[...] - m * sc

    @pl.when(i >= T)
    def _():
        o_ref[...] = jnp.maximum(z * sc_ref[...] + sh_ref[...], 0.0)


def _mmbn_relu(x, w_t, g, b, R):
    N, Cin = x.shape
    Cout = w_t.shape[1]
    T = N // R
    return pl.pallas_call(
        functools.partial(_mmbn_body, T, N),
        grid=(2 * T,),
        in_specs=[
            pl.BlockSpec((R, Cin), lambda i: (i % T, 0)),
            pl.BlockSpec((Cin, Cout), lambda i: (0, 0)),
            pl.BlockSpec((1, Cout), lambda i: (0, 0)),
            pl.BlockSpec((1, Cout), lambda i: (0, 0)),
        ],
        out_specs=pl.BlockSpec((R, Cout), lambda i: (i % T, 0)),
        out_shape=jax.ShapeDtypeStruct((N, Cout), F32),
        scratch_shapes=[pltpu.VMEM((1, Cout), F32)] * 4,
    )(x, w_t, g.reshape(1, -1), b.reshape(1, -1))


_Y0B = 128
_Y0_FULL = N0 // _Y0B
_Y0_TAIL = N0 - _Y0_FULL * _Y0B
_Y0_TAIL_W = _Y0_FULL % NW


def _y0_body(u_hbm, f0_hbm, cl_hbm, y0_hbm, idx_v, ug_v, f0_v,
             idx_t, ug_t, f0_t, sem):
    wid = lax.axis_index("s") * NC + lax.axis_index("c")

    def do_rows(base, nrows, idx_ref, ug_ref, f0_ref):
        pltpu.sync_copy(cl_hbm.at[pl.ds(base, nrows)], idx_ref)
        pltpu.async_copy(u_hbm.at[idx_ref], ug_ref, sem).wait()
        pltpu.sync_copy(f0_hbm.at[pl.ds(base, nrows)], f0_ref)

        def row(r, _):
            for j in range(C // 16):
                sl = pl.ds(j * 16, 16)
                a = f0_ref[r, sl] + ug_ref[r, sl]
                f0_ref[r, sl] = jnp.maximum(a, 0.0)
            return 0

        lax.fori_loop(0, nrows, row, 0)
        pltpu.sync_copy(f0_ref, y0_hbm.at[pl.ds(base, nrows)])

    nmine = (_Y0_FULL - wid + NW - 1) // NW

    def blk(t, _):
        base = pl.multiple_of((wid + t * NW) * _Y0B, _Y0B)
        do_rows(base, _Y0B, idx_v, ug_v, f0_v)
        return 0

    lax.fori_loop(0, nmine, blk, 0)

    @pl.when(wid == _Y0_TAIL_W)
    def _():
        do_rows(_Y0_FULL * _Y0B, _Y0_TAIL, idx_t, ug_t, f0_t)


def _y0_call(u, feat0, cluster):
    mesh = plsc.VectorSubcoreMesh(core_axis_name="c", subcore_axis_name="s", num_cores=NC, num_subcores=NS)
    return pl.kernel(
        _y0_body,
        out_type=jax.ShapeDtypeStruct((N0, C), F32),
        mesh=mesh,
        scratch_types=[
            pltpu.VMEM((_Y0B,), jnp.int32),
            pltpu.VMEM((_Y0B, C), F32),
            pltpu.VMEM((_Y0B, C), F32),
            pltpu.VMEM((_Y0_TAIL,), jnp.int32),
            pltpu.VMEM((_Y0_TAIL, C), F32),
            pltpu.VMEM((_Y0_TAIL, C), F32),
            pltpu.SemaphoreType.DMA,
        ],
    )(u, feat0, cluster)


_CB = 64
_NCB = N1 // _CB
_Y1K = 120
_Y1KP = 128
_C1 = 2 * C
_NG = _C1 // 16


def _y1_body(h_hbm, si_hbm, seg_hbm, ptr_hbm, f1_hbm, y1_hbm,
             idx_v, rows_v, seg_v, acc_v, fw_v, ow_v, pw_v, win_r, sem):
    wid = lax.axis_index("s") * NC + lax.axis_index("c")
    b0 = (_NCB * wid) // NW
    b1 = (_NCB * (wid + 1)) // NW
    c_lo = pl.multiple_of(b0 * _CB, _CB)
    c_hi = pl.multiple_of(jnp.where(wid == NW - 1, N1, b1 * _CB), 8)

    pltpu.sync_copy(ptr_hbm.at[pl.ds(c_lo, 16)], pw_v)
    rs = pw_v[pl.ds(0, 16)][0]
    pltpu.sync_copy(ptr_hbm.at[pl.ds(c_hi, 16)], pw_v)
    re = pw_v[pl.ds(0, 16)][0]

    for j in range(_NG):
        acc_v[pl.ds(j * 16, 16)] = jnp.zeros((16,), F32)
    win_r[0] = c_lo
    win_r[1] = c_lo
    pltpu.sync_copy(f1_hbm.at[pl.ds(c_lo, _CB)], fw_v)

    def finalize(cur):
        ws = win_r[0]
        rw = cur - ws
        rwf = cur - win_r[1]
        for j in range(_NG):
            sl = pl.ds(j * 16, 16)
            ow_v[rw, sl] = jnp.maximum(acc_v[sl] + fw_v[rwf, sl], 0.0)
            acc_v[sl] = jnp.zeros((16,), F32)

        @pl.when(rw == _CB - 1)
        def _():
            pltpu.sync_copy(ow_v, y1_hbm.at[pl.ds(pl.multiple_of(ws, _CB),
                                                  _CB)])
            nxt = pl.multiple_of(jnp.minimum(ws + _CB, N1 - _CB), 8)
            pltpu.sync_copy(f1_hbm.at[pl.ds(nxt, _CB)], fw_v)
            win_r[0] = ws + _CB
            win_r[1] = nxt

    def row_body(r, cur):
        sid = seg_v[pl.ds(r, 16)][0]
        changed = sid != cur

        @pl.when(changed)
        def _():
            finalize(cur)

        for j in range(_NG):
            sl = pl.ds(j * 16, 16)
            acc_v[sl] = jnp.maximum(acc_v[sl], rows_v[r, sl])
        return jnp.where(changed, sid, cur)

    def chunk(t, cur):
        row0 = rs + t * _Y1K
        abase = pl.multiple_of(
            jnp.minimum((row0 // 8) * 8, N0 - _Y1KP), 8)
        off0 = row0 - abase
        n_t = jnp.minimum(_Y1K, re - row0)
        pltpu.sync_copy(si_hbm.at[pl.ds(abase, _Y1KP)], idx_v)
        pltpu.async_copy(h_hbm.at[idx_v], rows_v, sem).wait()
        pltpu.sync_copy(seg_hbm.at[pl.ds(abase, _Y1KP)],
                        seg_v.at[pl.ds(0, _Y1KP)])
        return lax.fori_loop(off0, off0 + n_t, row_body, cur)

    nchunks = (re - rs + _Y1K - 1) // _Y1K
    cur = lax.fori_loop(0, nchunks, chunk, c_lo)
    finalize(cur)

    @pl.when(wid == NW - 1)
    def _():
        tail = N1 - _NCB * _CB
        pltpu.sync_copy(ow_v.at[pl.ds(0, tail)],
                        y1_hbm.at[pl.ds(_NCB * _CB, tail)])


def _y1_call(h, sorted_idx, seg_sorted, ptr_pad, feat1):
    mesh = plsc.VectorSubcoreMesh(core_axis_name="c", subcore_axis_name="s", num_cores=NC, num_subcores=NS)
    return pl.kernel(
        _y1_body,
        out_type=jax.ShapeDtypeStruct((N1, _C1), F32),
        mesh=mesh,
        scratch_types=[
            pltpu.VMEM((_Y1KP,), jnp.int32),
            pltpu.VMEM((_Y1KP, _C1), F32),
            pltpu.VMEM((_Y1KP + 16,), jnp.int32),
            pltpu.VMEM((_C1,), F32),
            pltpu.VMEM((_CB, _C1), F32),
            pltpu.VMEM((_CB, _C1), F32),
            pltpu.VMEM((16,), jnp.int32),
            pltpu.SMEM((2,), jnp.int32),
            pltpu.SemaphoreType.DMA,
        ],
    )(h, sorted_idx, seg_sorted, ptr_pad, feat1)


def kernel(coord0, feat0, offset0, coord1, feat1, offset1, cluster, idx_ptr,
           sorted_cluster_indices, W_u, b_u, bn_u_w, bn_u_b, W_g, bn_g_w,
           bn_g_b):
    u = _mmbn_relu(feat1, W_u.T, bn_u_w, bn_u_b, R=1000)
    h = _mmbn_relu(feat0, W_g.T, bn_g_w, bn_g_b, R=1000)

    y0 = _y0_call(u, feat0, cluster)

    seg_sorted = cluster[sorted_cluster_indices]
    ptr_pad = jnp.pad(idx_ptr, (0, 15))
    y1 = _y1_call(h, sorted_cluster_indices, seg_sorted, ptr_pad, feat1)

    return (coord0, y0, coord1, y1)


# speedup vs baseline: 1.5363x; 1.4760x over previous
import functools

import jax
import jax.numpy as jnp
from jax import lax
from jax.experimental import pallas as pl
from jax.experimental.pallas import tpu as pltpu
from jax.experimental.pallas import tpu_sc as plsc

N0 = 100000
N1 = 25000
C = 128
EPS = 1e-5

NC = 2
NS = 16
NW = NC * NS

F32 = jnp.float32


def _mmbn_body(T, N, x_ref, w_ref, g_ref, b_ref, o_ref, s_ref, ss_ref,
               sc_ref, sh_ref):
    i = pl.program_id(0)
    z = jnp.dot(x_ref[...], w_ref[...], preferred_element_type=F32)

    @pl.when(i == 0)
    def _():
        s_ref[...] = jnp.zeros_like(s_ref)
        ss_ref[...] = jnp.zeros_like(ss_ref)

    @pl.when(i < T)
    def _():
        s_ref[...] += jnp.sum(z, axis=0, keepdims=True)
        ss_ref[...] += jnp.sum(z * z, axis=0, keepdims=True)

    @pl.when(i == T)
    def _():
        m = s_ref[...] / N
        v = ss_ref[...] / N - m * m
        sc = g_ref[...] * lax.rsqrt(v + EPS)
        sc_ref[...] = sc
        sh_ref[...] = b_ref[...] - m * sc

    @pl.when(i >= T)
    def _():
        o_ref[...] = jnp.maximum(z * sc_ref[...] + sh_ref[...], 0.0)


def _mmbn_relu(x, w_t, g, b, R):
    N, Cin = x.shape
    Cout = w_t.shape[1]
    T = N // R
    return pl.pallas_call(
        functools.partial(_mmbn_body, T, N),
        grid=(2 * T,),
        in_specs=[
            pl.BlockSpec((R, Cin), lambda i: (i % T, 0)),
            pl.BlockSpec((Cin, Cout), lambda i: (0, 0)),
            pl.BlockSpec((1, Cout), lambda i: (0, 0)),
            pl.BlockSpec((1, Cout), lambda i: (0, 0)),
        ],
        out_specs=pl.BlockSpec((R, Cout), lambda i: (i % T, 0)),
        out_shape=jax.ShapeDtypeStruct((N, Cout), F32),
        scratch_shapes=[pltpu.VMEM((1, Cout), F32)] * 4,
    )(x, w_t, g.reshape(1, -1), b.reshape(1, -1))


_Y0B = 128
_Y0_FULL = N0 // _Y0B
_Y0_TAIL = N0 - _Y0_FULL * _Y0B
_Y0_TAIL_W = _Y0_FULL % NW


def _y0_body(u_hbm, f0_hbm, cl_hbm, y0_hbm,
             idx_a, ug_a, f0_a, idx_b, ug_b, f0_b,
             idx_t, ug_t, f0_t, sem_g, sem_f):
    wid = lax.axis_index("s") * NC + lax.axis_index("c")
    nmine = (_Y0_FULL - wid + NW - 1) // NW

    def base_of(t):
        return pl.multiple_of((wid + t * NW) * _Y0B, _Y0B)

    def start(t, idx_ref, ug_ref, f0_ref):
        base = base_of(t)
        pltpu.sync_copy(cl_hbm.at[pl.ds(base, _Y0B)], idx_ref)
        pltpu.async_copy(u_hbm.at[idx_ref], ug_ref, sem_g)
        pltpu.async_copy(f0_hbm.at[pl.ds(base, _Y0B)], f0_ref, sem_f)

    def finish(t, idx_ref, ug_ref, f0_ref):
        pltpu.make_async_copy(u_hbm.at[idx_ref], ug_ref, sem_g).wait()
        base = base_of(t)
        pltpu.make_async_copy(f0_hbm.at[pl.ds(base, _Y0B)], f0_ref,
                              sem_f).wait()

        def row(r, _):
            for j in range(C // 16):
                sl = pl.ds(j * 16, 16)
                a = f0_ref[r, sl] + ug_ref[r, sl]
                f0_ref[r, sl] = jnp.maximum(a, 0.0)
            return 0

        lax.fori_loop(0, _Y0B, row, 0)
        pltpu.sync_copy(f0_ref, y0_hbm.at[pl.ds(base, _Y0B)])

    start(0, idx_a, ug_a, f0_a)
    npairs = (nmine + 1) // 2

    def pair(p, _):
        t0 = 2 * p
        t1 = t0 + 1

        @pl.when(t1 < nmine)
        def _():
            start(t1, idx_b, ug_b, f0_b)

        finish(t0, idx_a, ug_a, f0_a)

        @pl.when(t1 < nmine)
        def _():
            @pl.when(t1 + 1 < nmine)
            def _():
                start(t1 + 1, idx_a, ug_a, f0_a)

            finish(t1, idx_b, ug_b, f0_b)

        return 0

    lax.fori_loop(0, npairs, pair, 0)

    @pl.when(wid == _Y0_TAIL_W)
    def _():
        base = _Y0_FULL * _Y0B
        pltpu.sync_copy(cl_hbm.at[pl.ds(base, _Y0_TAIL)], idx_t)
        pltpu.async_copy(u_hbm.at[idx_t], ug_t, sem_g).wait()
        pltpu.sync_copy(f0_hbm.at[pl.ds(base, _Y0_TAIL)], f0_t)

        def row(r, _):
            for j in range(C // 16):
                sl = pl.ds(j * 16, 16)
                a = f0_t[r, sl] + ug_t[r, sl]
                f0_t[r, sl] = jnp.maximum(a, 0.0)
            return 0

        lax.fori_loop(0, _Y0_TAIL, row, 0)
        pltpu.sync_copy(f0_t, y0_hbm.at[pl.ds(base, _Y0_TAIL)])


def _y0_call(u, feat0, cluster):
    mesh = plsc.VectorSubcoreMesh(core_axis_name="c", subcore_axis_name="s", num_cores=NC, num_subcores=NS)
    return pl.kernel(
        _y0_body,
        out_type=jax.ShapeDtypeStruct((N0, C), F32),
        mesh=mesh,
        scratch_types=[
            pltpu.VMEM((_Y0B,), jnp.int32),
            pltpu.VMEM((_Y0B, C), F32),
            pltpu.VMEM((_Y0B, C), F32),
            pltpu.VMEM((_Y0B,), jnp.int32),
            pltpu.VMEM((_Y0B, C), F32),
            pltpu.VMEM((_Y0B, C), F32),
            pltpu.VMEM((_Y0_TAIL,), jnp.int32),
            pltpu.VMEM((_Y0_TAIL, C), F32),
            pltpu.VMEM((_Y0_TAIL, C), F32),
            pltpu.SemaphoreType.DMA,
            pltpu.SemaphoreType.DMA,
        ],
    )(u, feat0, cluster)


_CB = 64
_NCB = N1 // _CB
_Y1K = 120
_Y1KP = 128
_C1 = 2 * C
_NG = _C1 // 16


_PTRW = 896


def _y1_body(h_hbm, si_hbm, ptr_hbm, f1_hbm, y1_hbm,
             idx_a, rows_a, idx_b, rows_b, ptr_v, acc_v, fw_v, ow_v,
             win_r, cur_r, sem_a, sem_b):
    wid = lax.axis_index("s") * NC + lax.axis_index("c")
    b0 = (_NCB * wid) // NW
    b1 = (_NCB * (wid + 1)) // NW
    c_lo = pl.multiple_of(b0 * _CB, _CB)
    c_hi = jnp.where(wid == NW - 1, N1, b1 * _CB)

    pltpu.sync_copy(ptr_hbm.at[pl.ds(c_lo, _PTRW)], ptr_v)
    rs = ptr_v[pl.ds(0, 16)][0]
    re = ptr_v[pl.ds(c_hi - c_lo, 16)][0]

    for j in range(_NG):
        acc_v[pl.ds(j * 16, 16)] = jnp.zeros((16,), F32)
    win_r[0] = c_lo
    win_r[1] = c_lo
    cur_r[0] = c_lo
    pltpu.sync_copy(f1_hbm.at[pl.ds(c_lo, _CB)], fw_v)

    def abase_of(t):
        row0 = rs + t * _Y1K
        return row0, pl.multiple_of(
            jnp.minimum((row0 // 8) * 8, N0 - _Y1KP), 8)

    def stage(t, idx_ref, rows_ref, sem):
        _, abase = abase_of(t)
        pltpu.sync_copy(si_hbm.at[pl.ds(abase, _Y1KP)], idx_ref)
        pltpu.async_copy(h_hbm.at[idx_ref], rows_ref, sem)

    def finalize(cur, acc):
        ws = win_r[0]
        rw = cur - ws
        rwf = cur - win_r[1]
        for j in range(_NG):
            sl = pl.ds(j * 16, 16)
            ow_v[rw, sl] = jnp.maximum(acc[j] + fw_v[rwf, sl], 0.0)

        @pl.when(rw == _CB - 1)
        def _():
            pltpu.sync_copy(ow_v, y1_hbm.at[pl.ds(pl.multiple_of(ws, _CB),
                                                  _CB)])
            nxt = pl.multiple_of(jnp.minimum(ws + _CB, N1 - _CB), 8)
            pltpu.sync_copy(f1_hbm.at[pl.ds(nxt, _CB)], fw_v)
            win_r[0] = ws + _CB
            win_r[1] = nxt

    def process(t, idx_ref, rows_ref, sem):
        row0, abase = abase_of(t)
        pltpu.make_async_copy(h_hbm.at[idx_ref], rows_ref, sem).wait()
        off0 = row0 - abase
        end = off0 + jnp.minimum(_Y1K, re - row0)
        acc0 = tuple(acc_v[pl.ds(j * 16, 16)] for j in range(_NG))
        cur0 = cur_r[0]
        pe0 = ptr_v[pl.ds(cur0 + 1 - c_lo, 16)][0]

        def rbody(r, st):
            cur, pe = st[0], st[1]
            acc = tuple(
                jnp.maximum(st[2 + j], rows_ref[r, pl.ds(j * 16, 16)])
                for j in range(_NG))
            fin = (abase + r + 1) == pe

            @pl.when(fin)
            def _():
                finalize(cur, acc)

            acc = tuple(jnp.where(fin, 0.0, a) for a in acc)
            ncur = jnp.where(fin, cur + 1, cur)
            npe = jnp.where(
                fin, ptr_v[pl.ds(ncur + 1 - c_lo, 16)][0], pe)
            return (ncur, npe) + acc

        st = lax.fori_loop(off0, end, rbody, (cur0, pe0) + acc0)
        cur_r[0] = st[0]
        for j in range(_NG):
            acc_v[pl.ds(j * 16, 16)] = st[2 + j]

    nchunks = (re - rs + _Y1K - 1) // _Y1K
    stage(0, idx_a, rows_a, sem_a)
    npairs = (nchunks + 1) // 2

    def pair(p, _):
        t0 = 2 * p
        t1 = t0 + 1

        @pl.when(t1 < nchunks)
        def _():
            stage(t1, idx_b, rows_b, sem_b)

        process(t0, idx_a, rows_a, sem_a)

        @pl.when(t1 < nchunks)
        def _():
            @pl.when(t1 + 1 < nchunks)
            def _():
                stage(t1 + 1, idx_a, rows_a, sem_a)

            process(t1, idx_b, rows_b, sem_b)

        return 0

    lax.fori_loop(0, npairs, pair, 0)

    @pl.when(wid == NW - 1)
    def _():
        tail = N1 - _NCB * _CB
        pltpu.sync_copy(ow_v.at[pl.ds(0, tail)],
                        y1_hbm.at[pl.ds(_NCB * _CB, tail)])


def _y1_call(h, sorted_idx, ptr_pad, feat1):
    mesh = plsc.VectorSubcoreMesh(core_axis_name="c", subcore_axis_name="s", num_cores=NC, num_subcores=NS)
    return pl.kernel(
        _y1_body,
        out_type=jax.ShapeDtypeStruct((N1, _C1), F32),
        mesh=mesh,
        scratch_types=[
            pltpu.VMEM((_Y1KP,), jnp.int32),
            pltpu.VMEM((_Y1KP, _C1), F32),
            pltpu.VMEM((_Y1KP,), jnp.int32),
            pltpu.VMEM((_Y1KP, _C1), F32),
            pltpu.VMEM((_PTRW,), jnp.int32),
            pltpu.VMEM((_C1,), F32),
            pltpu.VMEM((_CB, _C1), F32),
            pltpu.VMEM((_CB, _C1), F32),
            pltpu.SMEM((2,), jnp.int32),
            pltpu.SMEM((1,), jnp.int32),
            pltpu.SemaphoreType.DMA,
            pltpu.SemaphoreType.DMA,
        ],
    )(h, sorted_idx, ptr_pad, feat1)


def kernel(coord0, feat0, offset0, coord1, feat1, offset1, cluster, idx_ptr,
           sorted_cluster_indices, W_u, b_u, bn_u_w, bn_u_b, W_g, bn_g_w,
           bn_g_b):
    u = _mmbn_relu(feat1, W_u.T, bn_u_w, bn_u_b, R=1000)
    h = _mmbn_relu(feat0, W_g.T, bn_g_w, bn_g_b, R=1000)

    y0 = _y0_call(u, feat0, cluster)

    ptr_pad = jnp.pad(idx_ptr, (0, 23))
    y1 = _y1_call(h, sorted_cluster_indices, ptr_pad, feat1)

    return (coord0, y0, coord1, y1)
